# Initial kernel scaffold; baseline (speedup 1.0000x reference)
import jax
import jax.numpy as jnp
from jax.experimental import pallas as pl


def kernel(x):
    raise NotImplementedError("write your pallas kernel here")


# speedup vs baseline: 2.0468x; 2.0468x over previous
import functools

import jax
import jax.numpy as jnp
from jax import lax
from jax.experimental import pallas as pl
from jax.experimental.pallas import tpu as pltpu
from jax.experimental.pallas import tpu_sc as plsc

_B, _C, _H, _W = 4, 96, 384, 384
_PAD = 1
_LANES = 16

_CGROUPS = 8
_CPG = _C // _CGROUPS

_RS = 96
_STRIPS = ((0, 0, 97), (96, 95, 98), (192, 191, 98), (288, 287, 97))
_IN_ROWS = 98


def _sc_gather(x_hbm, g_hbm, out_hbm, idx_v, in_v, out_v):
    info = plsc.get_sparse_core_info()
    nc = info.num_cores
    wid = lax.axis_index("s") * nc + lax.axis_index("c")
    b = wid // _CGROUPS
    cbase = (wid % _CGROUPS) * _CPG

    n_chunks = (_RS * _W) // _LANES

    for r0, a, nrows in _STRIPS:
        pltpu.sync_copy(g_hbm.at[pl.ds(b * _H * _W + r0 * _W, _RS * _W)],
                        idx_v)
        base = a * _W

        def chan_body(j, carry, r0=r0, a=a, nrows=nrows, base=base):
            c = cbase + j
            img = (b * _C + c) * _H * _W
            pltpu.sync_copy(x_hbm.at[pl.ds(img + a * _W, nrows * _W)],
                            in_v.at[pl.ds(0, nrows * _W)])

            def gather_body(i, carry2):
                off = i * _LANES
                lidx = idx_v[pl.ds(off, _LANES)] - base
                out_v[pl.ds(off, _LANES)] = plsc.load_gather(in_v, [lidx])
                return carry2

            lax.fori_loop(0, n_chunks, gather_body, 0, unroll=4)
            pltpu.sync_copy(out_v, out_hbm.at[pl.ds(img + r0 * _W, _RS * _W)])
            return carry

        lax.fori_loop(0, _CPG, chan_body, 0)


@jax.jit
def kernel(x):
    B, C, H, W = x.shape

    kh = jax.random.fold_in(jax.random.key(0), 1)
    kw = jax.random.fold_in(jax.random.key(0), 2)
    dh = jax.random.randint(kh, (B, 1, H, W), -_PAD, _PAD + 1)
    dw = jax.random.randint(kw, (B, 1, H, W), -_PAD, _PAD + 1)
    row = jnp.arange(H)[None, None, :, None] + dh
    col = jnp.arange(W)[None, None, None, :] + dw
    row = (H - 1) - jnp.abs((H - 1) - jnp.abs(row))
    col = (W - 1) - jnp.abs((W - 1) - jnp.abs(col))
    g = (row * W + col).astype(jnp.int32).reshape(B * H * W)

    sc = functools.partial(
        pl.kernel,
        out_type=jax.ShapeDtypeStruct((B * C * H * W,), jnp.float32),
        mesh=plsc.VectorSubcoreMesh(core_axis_name="c", subcore_axis_name="s"),
        compiler_params=pltpu.CompilerParams(needs_layout_passes=False),
        scratch_types=[
            pltpu.VMEM((_RS * _W,), jnp.int32),
            pltpu.VMEM((_IN_ROWS * _W,), jnp.float32),
            pltpu.VMEM((_RS * _W,), jnp.float32),
        ],
    )(_sc_gather)
    out = sc(x.reshape(B * C * H * W), g)
    return out.reshape(B, C, H, W)


# speedup vs baseline: 5.2325x; 2.5564x over previous
import functools

import jax
import jax.numpy as jnp
from jax import lax
from jax.experimental import pallas as pl
from jax.experimental.pallas import tpu as pltpu
from jax.experimental.pallas import tpu_sc as plsc

_B, _C, _H, _W = 4, 96, 384, 384
_PAD = 1
_LANES = 16

_CGROUPS = 8
_CPG = _C // _CGROUPS

_RS = 64
_STRIPS = tuple(
    (r0, max(0, r0 - 1), min(_H - 1, r0 + _RS) - max(0, r0 - 1) + 1)
    for r0 in range(0, _H, _RS)
)
_IN_ROWS = _RS + 2


def _sc_gather(x_hbm, g_hbm, out_hbm, idx_v, in_v0, in_v1, out_v0, out_v1,
               sems):
    in_bufs = (in_v0, in_v1)
    out_bufs = (out_v0, out_v1)
    info = plsc.get_sparse_core_info()
    nc = info.num_cores
    wid = lax.axis_index("s") * nc + lax.axis_index("c")
    b = wid // _CGROUPS
    cbase = (wid % _CGROUPS) * _CPG

    out_cp = [None, None]
    for r0, a, nrows in _STRIPS:
        base = a * _W

        def img_off(j):
            return (b * _C + (cbase + j)) * _H * _W

        in_cp = [None, None]
        in_cp[0] = pltpu.async_copy(
            x_hbm.at[pl.ds(img_off(0) + a * _W, nrows * _W)],
            in_bufs[0].at[pl.ds(0, nrows * _W)], sems.at[0])
        pltpu.sync_copy(g_hbm.at[pl.ds(b * _H * _W + r0 * _W, _RS * _W)],
                        idx_v)

        for j in range(_CPG):
            cur = j & 1
            nxt = cur ^ 1
            if j + 1 < _CPG:
                in_cp[nxt] = pltpu.async_copy(
                    x_hbm.at[pl.ds(img_off(j + 1) + a * _W, nrows * _W)],
                    in_bufs[nxt].at[pl.ds(0, nrows * _W)], sems.at[1 + nxt])
            in_cp[cur].wait()
            if out_cp[cur] is not None:
                out_cp[cur].wait()

            @plsc.parallel_loop(0, _RS * _W, step=_LANES, unroll=8)
            def gather_body(off, cur=cur, base=base):
                lidx = idx_v[pl.ds(off, _LANES)] - base
                out_bufs[cur][pl.ds(off, _LANES)] = plsc.load_gather(
                    in_bufs[cur], [lidx])

            out_cp[cur] = pltpu.async_copy(
                out_bufs[cur],
                out_hbm.at[pl.ds(img_off(j) + r0 * _W, _RS * _W)],
                sems.at[3 + cur])
    for cp in out_cp:
        if cp is not None:
            cp.wait()


@jax.jit
def kernel(x):
    B, C, H, W = x.shape

    kh = jax.random.fold_in(jax.random.key(0), 1)
    kw = jax.random.fold_in(jax.random.key(0), 2)
    dh = jax.random.randint(kh, (B, 1, H, W), -_PAD, _PAD + 1)
    dw = jax.random.randint(kw, (B, 1, H, W), -_PAD, _PAD + 1)
    row = jnp.arange(H)[None, None, :, None] + dh
    col = jnp.arange(W)[None, None, None, :] + dw
    row = (H - 1) - jnp.abs((H - 1) - jnp.abs(row))
    col = (W - 1) - jnp.abs((W - 1) - jnp.abs(col))
    g = (row * W + col).astype(jnp.int32).reshape(B * H * W)

    sc = functools.partial(
        pl.kernel,
        out_type=jax.ShapeDtypeStruct((B * C * H * W,), jnp.float32),
        mesh=plsc.VectorSubcoreMesh(core_axis_name="c", subcore_axis_name="s"),
        compiler_params=pltpu.CompilerParams(needs_layout_passes=False),
        scratch_types=[
            pltpu.VMEM((_RS * _W,), jnp.int32),
            pltpu.VMEM((_IN_ROWS * _W,), jnp.float32),
            pltpu.VMEM((_IN_ROWS * _W,), jnp.float32),
            pltpu.VMEM((_RS * _W,), jnp.float32),
            pltpu.VMEM((_RS * _W,), jnp.float32),
            pltpu.SemaphoreType.DMA((5,)),
        ],
    )(_sc_gather)
    out = sc(x.reshape(B * C * H * W), g)
    return out.reshape(B, C, H, W)


# speedup vs baseline: 8.2522x; 1.5771x over previous
import functools

import jax
import jax.numpy as jnp
from jax import lax
from jax.experimental import pallas as pl
from jax.experimental.pallas import tpu as pltpu
from jax.experimental.pallas import tpu_sc as plsc

_B, _C, _H, _W = 4, 96, 384, 384
_PAD = 1
_LANES = 16

_CGROUPS = 8
_CPG = _C // _CGROUPS

_RS = 48
_STRIPS = tuple(
    (r0, max(0, r0 - 8), min(_H, r0 + _RS + 8) - max(0, r0 - 8))
    for r0 in range(0, _H, _RS)
)
_IN_ROWS = _RS + 16


def _sc_gather(x_hbm, m_hbm, out_hbm, m_v, in_v0, in_v1, out_v0, out_v1,
               sems):
    in_bufs = (in_v0, in_v1)
    out_bufs = (out_v0, out_v1)
    info = plsc.get_sparse_core_info()
    nc = info.num_cores
    wid = lax.axis_index("s") * nc + lax.axis_index("c")
    b = wid // _CGROUPS
    cbase = (wid % _CGROUPS) * _CPG

    out_cp = [None, None]
    for r0, st, nst in _STRIPS:

        def row0(j):
            return (b * _C + (cbase + j)) * _H

        in_cp = [None, None]
        in_cp[0] = pltpu.async_copy(
            x_hbm.at[pl.ds(row0(0) + st, nst), :],
            in_bufs[0].at[pl.ds(0, nst), :], sems.at[0])
        pltpu.sync_copy(m_hbm.at[pl.ds(b * _H + r0, _RS), :], m_v)

        for j in range(_CPG):
            cur = j & 1
            nxt = cur ^ 1
            if j + 1 < _CPG:
                in_cp[nxt] = pltpu.async_copy(
                    x_hbm.at[pl.ds(row0(j + 1) + st, nst), :],
                    in_bufs[nxt].at[pl.ds(0, nst), :], sems.at[1 + nxt])
            in_cp[cur].wait()
            if out_cp[cur] is not None:
                out_cp[cur].wait()

            def row_body(h, carry, cur=cur, st=st):

                @plsc.parallel_loop(0, _W, step=_LANES, unroll=4)
                def gather_body(wb):
                    mv = m_v[h, pl.ds(wb, _LANES)]
                    lrow = (mv >> 9) - st
                    lcol = lax.bitwise_and(mv, 511)
                    out_bufs[cur][h, pl.ds(wb, _LANES)] = plsc.load_gather(
                        in_bufs[cur], [lrow, lcol])

                return carry

            lax.fori_loop(0, _RS, row_body, 0)
            out_cp[cur] = pltpu.async_copy(
                out_bufs[cur],
                out_hbm.at[pl.ds(row0(j) + r0, _RS), :],
                sems.at[3 + cur])
    for cp in out_cp:
        if cp is not None:
            cp.wait()


@jax.jit
def kernel(x):
    B, C, H, W = x.shape

    kh = jax.random.fold_in(jax.random.key(0), 1)
    kw = jax.random.fold_in(jax.random.key(0), 2)
    dh = jax.random.randint(kh, (B, 1, H, W), -_PAD, _PAD + 1)
    dw = jax.random.randint(kw, (B, 1, H, W), -_PAD, _PAD + 1)
    row = jnp.arange(H)[None, None, :, None] + dh
    col = jnp.arange(W)[None, None, None, :] + dw
    row = (H - 1) - jnp.abs((H - 1) - jnp.abs(row))
    col = (W - 1) - jnp.abs((W - 1) - jnp.abs(col))
    m = (row * 512 + col).astype(jnp.int32).reshape(B * H, W)

    sc = functools.partial(
        pl.kernel,
        out_type=jax.ShapeDtypeStruct((B * C * H, W), jnp.float32),
        mesh=plsc.VectorSubcoreMesh(core_axis_name="c", subcore_axis_name="s"),
        compiler_params=pltpu.CompilerParams(needs_layout_passes=False),
        scratch_types=[
            pltpu.VMEM((_RS, _W), jnp.int32),
            pltpu.VMEM((_IN_ROWS, _W), jnp.float32),
            pltpu.VMEM((_IN_ROWS, _W), jnp.float32),
            pltpu.VMEM((_RS, _W), jnp.float32),
            pltpu.VMEM((_RS, _W), jnp.float32),
            pltpu.SemaphoreType.DMA((5,)),
        ],
    )(_sc_gather)
    out = sc(x.reshape(B * C * H, W), m)
    return out.reshape(B, C, H, W)


# speedup vs baseline: 11.5350x; 1.3978x over previous
import functools

import jax
import jax.numpy as jnp
from jax import lax
from jax.experimental import pallas as pl
from jax.experimental.pallas import tpu as pltpu
from jax.experimental.pallas import tpu_sc as plsc

_B, _C, _H, _W = 4, 96, 384, 384
_PAD = 1
_LANES = 16

_CGROUPS = 8
_CPG = _C // _CGROUPS

_RS = 48
_HALO = 8
_STRIPS = tuple(
    (r0, max(0, r0 - _HALO), min(_H, r0 + _RS + _HALO) - max(0, r0 - _HALO))
    for r0 in range(0, _H, _RS)
)
_IN_ROWS = _RS + 2 * _HALO


def _sc_gather(x_hbm, m_hbm, dummy_hbm, out_hbm, m_v, in_v0, in_v1, out_v0,
               out_v1, sems):
    in_bufs = (in_v0, in_v1)
    out_bufs = (out_v0, out_v1)
    info = plsc.get_sparse_core_info()
    nc = info.num_cores
    wid = lax.axis_index("s") * nc + lax.axis_index("c")
    b = wid // _CGROUPS
    cbase = (wid % _CGROUPS) * _CPG

    out_cp = [None, None]
    for r0, st, nst in _STRIPS:

        def row0(j):
            return (b * _C + (cbase + j)) * _H

        def issue_rows(j, buf, sem_idx, st=st, nst=nst):
            base = row0(j) + st

            def body(k, carry):
                pltpu.async_copy(x_hbm.at[base + k],
                                 in_bufs[buf].at[pl.ds(k * _W, _W)],
                                 sems.at[sem_idx])
                return carry

            lax.fori_loop(0, nst, body, 0)

        issue_rows(0, 0, 0)
        pltpu.sync_copy(m_hbm.at[pl.ds(b * _H + r0, _RS), :], m_v)

        for j in range(_CPG):
            cur = j & 1
            nxt = cur ^ 1
            if j + 1 < _CPG:
                issue_rows(j + 1, nxt, nxt)
            pltpu.make_async_copy(
                dummy_hbm.at[pl.ds(0, nst * _W)],
                in_bufs[cur].at[pl.ds(0, nst * _W)],
                sems.at[cur]).wait()
            if out_cp[cur] is not None:
                out_cp[cur].wait()

            def row_body(h, carry, cur=cur):

                @plsc.parallel_loop(0, _W, step=_LANES, unroll=8)
                def gather_body(wb):
                    lidx = m_v[h, pl.ds(wb, _LANES)]
                    out_bufs[cur][h, pl.ds(wb, _LANES)] = plsc.load_gather(
                        in_bufs[cur], [lidx])

                return carry

            lax.fori_loop(0, _RS, row_body, 0)
            out_cp[cur] = pltpu.async_copy(
                out_bufs[cur],
                out_hbm.at[pl.ds(row0(j) + r0, _RS), :],
                sems.at[2 + cur])
    for cp in out_cp:
        if cp is not None:
            cp.wait()


@jax.jit
def kernel(x):
    B, C, H, W = x.shape

    kh = jax.random.fold_in(jax.random.key(0), 1)
    kw = jax.random.fold_in(jax.random.key(0), 2)
    dh = jax.random.randint(kh, (B, 1, H, W), -_PAD, _PAD + 1)
    dw = jax.random.randint(kw, (B, 1, H, W), -_PAD, _PAD + 1)
    row = jnp.arange(H)[None, None, :, None] + dh
    col = jnp.arange(W)[None, None, None, :] + dw
    row = (H - 1) - jnp.abs((H - 1) - jnp.abs(row))
    col = (W - 1) - jnp.abs((W - 1) - jnp.abs(col))
    st_h = jnp.maximum(0, (jnp.arange(H) // _RS) * _RS - _HALO)
    m = ((row - st_h[None, None, :, None]) * W + col).astype(jnp.int32)
    m = m.reshape(B * H, W)

    sc = functools.partial(
        pl.kernel,
        out_type=jax.ShapeDtypeStruct((B * C * H, W), jnp.float32),
        mesh=plsc.VectorSubcoreMesh(core_axis_name="c", subcore_axis_name="s"),
        compiler_params=pltpu.CompilerParams(needs_layout_passes=False),
        scratch_types=[
            pltpu.VMEM((_RS, _W), jnp.int32),
            pltpu.VMEM((_IN_ROWS * _W,), jnp.float32),
            pltpu.VMEM((_IN_ROWS * _W,), jnp.float32),
            pltpu.VMEM((_RS, _W), jnp.float32),
            pltpu.VMEM((_RS, _W), jnp.float32),
            pltpu.SemaphoreType.DMA((4,)),
        ],
    )(_sc_gather)
    dummy = jnp.zeros((_IN_ROWS * _W,), jnp.float32)
    out = sc(x.reshape(B * C * H, W), m, dummy)
    return out.reshape(B, C, H, W)


# speedup vs baseline: 14.0093x; 1.2145x over previous
import functools

import jax
import jax.numpy as jnp
from jax import lax
from jax.experimental import pallas as pl
from jax.experimental.pallas import tpu as pltpu
from jax.experimental.pallas import tpu_sc as plsc

_B, _C, _H, _W = 4, 96, 384, 384
_PAD = 1
_LANES = 16

_CGROUPS = 8
_CPG = _C // _CGROUPS

_RS = 48
_HALO = 8
_STRIPS = tuple(
    (r0, max(0, r0 - _HALO), min(_H, r0 + _RS + _HALO) - max(0, r0 - _HALO))
    for r0 in range(0, _H, _RS)
)
_IN_ROWS = _RS + 2 * _HALO


def _sc_gather(x_hbm, m_hbm, dummy_hbm, out_hbm, m_v, in_v0, in_v1, out_v0,
               out_v1, sems):
    in_bufs = (in_v0, in_v1)
    out_bufs = (out_v0, out_v1)
    info = plsc.get_sparse_core_info()
    nc = info.num_cores
    wid = lax.axis_index("s") * nc + lax.axis_index("c")
    b = wid // _CGROUPS
    cbase = (wid % _CGROUPS) * _CPG

    out_cp = [None, None]
    for r0, st, nst in _STRIPS:

        def row0(j):
            return (b * _C + (cbase + j)) * _H

        def issue_rows(j, buf, sem_idx, st=st, nst=nst):
            base = row0(j) + st

            def body(k, carry):
                pltpu.async_copy(x_hbm.at[base + k],
                                 in_bufs[buf].at[pl.ds(k * _W, _W)],
                                 sems.at[sem_idx])
                return carry

            lax.fori_loop(0, nst, body, 0)

        issue_rows(0, 0, 0)
        pltpu.sync_copy(m_hbm.at[pl.ds(b * _H + r0, _RS), :], m_v)

        for j in range(_CPG):
            cur = j & 1
            nxt = cur ^ 1
            if j + 1 < _CPG:
                issue_rows(j + 1, nxt, nxt)
            pltpu.make_async_copy(
                dummy_hbm.at[pl.ds(0, nst * _W)],
                in_bufs[cur].at[pl.ds(0, nst * _W)],
                sems.at[cur]).wait()
            if out_cp[cur] is not None:
                out_cp[cur].wait()

            def row_body(h, carry, cur=cur):

                @plsc.parallel_loop(0, _W, step=_LANES, unroll=8)
                def gather_body(wb):
                    lidx = m_v[h, pl.ds(wb, _LANES)]
                    out_bufs[cur][h, pl.ds(wb, _LANES)] = plsc.load_gather(
                        in_bufs[cur], [lidx])

                return carry

            lax.fori_loop(0, _RS, row_body, 0)
            out_cp[cur] = pltpu.async_copy(
                out_bufs[cur],
                out_hbm.at[pl.ds(row0(j) + r0, _RS), :],
                sems.at[2 + cur])
    for cp in out_cp:
        if cp is not None:
            cp.wait()


_M_CACHE = [None]


def _index_map(B, H, W):
    if _M_CACHE[0] is not None:
        return _M_CACHE[0]
    with jax.ensure_compile_time_eval():
        return _index_map_eager(B, H, W)


def _index_map_eager(B, H, W):
    kh = jax.random.fold_in(jax.random.key(0), 1)
    kw = jax.random.fold_in(jax.random.key(0), 2)
    dh = jax.random.randint(kh, (B, 1, H, W), -_PAD, _PAD + 1)
    dw = jax.random.randint(kw, (B, 1, H, W), -_PAD, _PAD + 1)
    row = jnp.arange(H)[None, None, :, None] + dh
    col = jnp.arange(W)[None, None, None, :] + dw
    row = (H - 1) - jnp.abs((H - 1) - jnp.abs(row))
    col = (W - 1) - jnp.abs((W - 1) - jnp.abs(col))
    st_h = jnp.maximum(0, (jnp.arange(H) // _RS) * _RS - _HALO)
    m = ((row - st_h[None, None, :, None]) * W + col).astype(jnp.int32)
    import numpy as np
    _M_CACHE[0] = np.asarray(m).reshape(B * H, W)
    return _M_CACHE[0]


@jax.jit
def kernel(x):
    B, C, H, W = x.shape
    m = jnp.asarray(_index_map(B, H, W))

    sc = functools.partial(
        pl.kernel,
        out_type=jax.ShapeDtypeStruct((B * C * H, W), jnp.float32),
        mesh=plsc.VectorSubcoreMesh(core_axis_name="c", subcore_axis_name="s"),
        compiler_params=pltpu.CompilerParams(needs_layout_passes=False),
        scratch_types=[
            pltpu.VMEM((_RS, _W), jnp.int32),
            pltpu.VMEM((_IN_ROWS * _W,), jnp.float32),
            pltpu.VMEM((_IN_ROWS * _W,), jnp.float32),
            pltpu.VMEM((_RS, _W), jnp.float32),
            pltpu.VMEM((_RS, _W), jnp.float32),
            pltpu.SemaphoreType.DMA((4,)),
        ],
    )(_sc_gather)
    dummy = jnp.zeros((_IN_ROWS * _W,), jnp.float32)
    out = sc(x.reshape(B * C * H, W), m, dummy)
    return out.reshape(B, C, H, W)


# speedup vs baseline: 17.2340x; 1.2302x over previous
import functools

import jax
import jax.numpy as jnp
import numpy as np
from jax import lax
from jax.experimental import pallas as pl
from jax.experimental.pallas import tpu as pltpu
from jax.experimental.pallas import tpu_sc as plsc

_B, _C, _H, _W = 4, 96, 384, 384
_PAD = 1
_LANES = 16

_CGROUPS = 8
_CPG = _C // _CGROUPS

_RS = 64
_HALO = 8
_STRIPS = tuple(
    (r0, max(0, r0 - _HALO), min(_H, r0 + _RS + _HALO) - max(0, r0 - _HALO))
    for r0 in range(0, _H, _RS)
)
_IN_ROWS = _RS + 2 * _HALO


def _sc_gather(x_hbm, m_hbm, dummy_hbm, dummy_m_hbm, out_hbm, m_v, in_v0,
               in_v1, out_v0, out_v1, sems):
    in_bufs = (in_v0, in_v1)
    out_bufs = (out_v0, out_v1)
    info = plsc.get_sparse_core_info()
    nc = info.num_cores
    wid = lax.axis_index("s") * nc + lax.axis_index("c")
    b = wid // _CGROUPS
    cbase = (wid % _CGROUPS) * _CPG

    out_cp = [None, None]
    for r0, st, nst in _STRIPS:

        def row0(j):
            return (b * _C + (cbase + j)) * _H

        def issue_rows(j, buf, sem_idx, st=st, nst=nst):
            base = row0(j) + st

            def body(k, carry):
                pltpu.async_copy(x_hbm.at[base + k],
                                 in_bufs[buf].at[pl.ds(k * _W, _W)],
                                 sems.at[sem_idx])
                return carry

            lax.fori_loop(0, nst, body, 0)

        issue_rows(0, 0, 0)
        m_base = b * (_H // 2) + r0 // 2

        def m_body(k, carry):
            pltpu.async_copy(m_hbm.at[m_base + k],
                             m_v.at[pl.ds(k * _W, _W)], sems.at[4])
            return carry

        lax.fori_loop(0, _RS // 2, m_body, 0)
        pltpu.make_async_copy(dummy_m_hbm, m_v, sems.at[4]).wait()

        for j in range(_CPG):
            cur = j & 1
            nxt = cur ^ 1
            if j + 1 < _CPG:
                issue_rows(j + 1, nxt, nxt)
            pltpu.make_async_copy(
                dummy_hbm.at[pl.ds(0, nst * _W)],
                in_bufs[cur].at[pl.ds(0, nst * _W)],
                sems.at[cur]).wait()
            if out_cp[cur] is not None:
                out_cp[cur].wait()

            @plsc.parallel_loop(0, (_RS // 2) * _W, step=_LANES, unroll=4)
            def gather_body(off, cur=cur):
                h = off // (_W // 2)
                rem = off - h * (_W // 2)
                v = m_v[pl.ds(off, _LANES)]
                lo = lax.bitwise_and(v, 0xFFFF)
                hi = lax.shift_right_logical(v, 16)
                out_bufs[cur][h, pl.ds(2 * rem, _LANES)] = (
                    plsc.load_gather(in_bufs[cur], [lo]))
                out_bufs[cur][h, pl.ds(2 * rem + _LANES, _LANES)] = (
                    plsc.load_gather(in_bufs[cur], [hi]))
            out_cp[cur] = pltpu.async_copy(
                out_bufs[cur],
                out_hbm.at[pl.ds(row0(j) + r0, _RS), :],
                sems.at[2 + cur])
    for cp in out_cp:
        if cp is not None:
            cp.wait()


_M_CACHE = [None]


def _index_map(B, H, W):
    if _M_CACHE[0] is not None:
        return _M_CACHE[0]
    with jax.ensure_compile_time_eval():
        kh = jax.random.fold_in(jax.random.key(0), 1)
        kw = jax.random.fold_in(jax.random.key(0), 2)
        dh = jax.random.randint(kh, (B, 1, H, W), -_PAD, _PAD + 1)
        dw = jax.random.randint(kw, (B, 1, H, W), -_PAD, _PAD + 1)
        row = jnp.arange(H)[None, None, :, None] + dh
        col = jnp.arange(W)[None, None, None, :] + dw
        row = (H - 1) - jnp.abs((H - 1) - jnp.abs(row))
        col = (W - 1) - jnp.abs((W - 1) - jnp.abs(col))
        st_h = jnp.maximum(0, (jnp.arange(H) // _RS) * _RS - _HALO)
        mloc = ((row - st_h[None, None, :, None]) * W + col).astype(jnp.int32)
    mloc = np.asarray(mloc).reshape(B * H, W)
    r = mloc.reshape(B * H, W // 32, 2, _LANES)
    packed = (r[:, :, 0, :] | (r[:, :, 1, :] << 16)).astype(np.int32)
    _M_CACHE[0] = packed.reshape(B * H // 2, W)
    return _M_CACHE[0]


@jax.jit
def kernel(x):
    B, C, H, W = x.shape
    m = jnp.asarray(_index_map(B, H, W))

    sc = functools.partial(
        pl.kernel,
        out_type=jax.ShapeDtypeStruct((B * C * H, W), jnp.float32),
        mesh=plsc.VectorSubcoreMesh(core_axis_name="c", subcore_axis_name="s"),
        compiler_params=pltpu.CompilerParams(needs_layout_passes=False),
        scratch_types=[
            pltpu.VMEM(((_RS // 2) * _W,), jnp.int32),
            pltpu.VMEM((_IN_ROWS * _W,), jnp.float32),
            pltpu.VMEM((_IN_ROWS * _W,), jnp.float32),
            pltpu.VMEM((_RS, _W), jnp.float32),
            pltpu.VMEM((_RS, _W), jnp.float32),
            pltpu.SemaphoreType.DMA((4,)),
        ],
    )(_sc_gather)
    dummy = jnp.zeros((_IN_ROWS * _W,), jnp.float32)
    dummy_m = jnp.zeros(((_RS // 2) * _W,), jnp.int32)
    out = sc(x.reshape(B * C * H, W), m, dummy, dummy_m)
    return out.reshape(B, C, H, W)
